# Initial kernel scaffold; baseline (speedup 1.0000x reference)
import jax
import jax.numpy as jnp
from jax.experimental import pallas as pl


def kernel(h, x, c, graph, Wz, bz, Wr, br, Wq, bq):
    raise NotImplementedError("write your pallas kernel here")


# speedup vs baseline: 8.1306x; 8.1306x over previous
import functools

import jax
import jax.numpy as jnp
from jax import lax
from jax.experimental import pallas as pl
from jax.experimental.pallas import tpu as pltpu
from jax.experimental.pallas import tpu_sc as plsc

_N = 10000
_K = 32
_HID = 64
_IN = 128

_NC, _NS = 2, 16
_NW = _NC * _NS
_NPW = 320
_NPAD = _NW * _NPW
_CPC = 4
_CK = _CPC * _K
_NCH = _NPW // _CPC


def _gather_max(table, gidx, D):
    mesh = plsc.VectorSubcoreMesh(
        core_axis_name="c", subcore_axis_name="s",
        num_cores=_NC, num_subcores=_NS)

    @functools.partial(
        pl.kernel,
        out_type=jax.ShapeDtypeStruct((_NPAD, D), jnp.float32),
        mesh=mesh,
        compiler_params=pltpu.CompilerParams(use_tc_tiling_on_sc=False),
        scratch_types=[
            pltpu.VMEM((_NCH, _CK), jnp.int32),
            pltpu.VMEM((2, _CK, D), jnp.float32),
            pltpu.VMEM((_NPW, D), jnp.float32),
            pltpu.SemaphoreType.DMA,
            pltpu.SemaphoreType.DMA,
        ],
    )
    def gmax(table_hbm, gidx_hbm, out_hbm, gidx_v, rows_v, out_v, sem0, sem1):
        wid = lax.axis_index("s") * _NC + lax.axis_index("c")
        pltpu.sync_copy(gidx_hbm.at[wid], gidx_v)
        sems = (sem0, sem1)

        def dma(g, b):
            return pltpu.make_async_copy(
                table_hbm.at[gidx_v.at[g]], rows_v.at[b], sems[b])

        dma(0, 0).start()
        dma(1, 1).start()

        def chunk_body(g, b):
            dma(g, b).wait()
            for c in range(_CPC):
                row0 = c * _K
                for dk in range(D // 16):
                    ds = pl.ds(dk * 16, 16)
                    acc = rows_v[b, row0, ds]
                    for k in range(1, _K):
                        acc = jnp.maximum(acc, rows_v[b, row0 + k, ds])
                    out_v[g * _CPC + c, ds] = acc

            @pl.when(g + 2 < _NCH)
            def _():
                dma(g + 2, b).start()

        def body(g2, carry):
            chunk_body(g2 * 2, 0)
            chunk_body(g2 * 2 + 1, 1)
            return carry

        lax.fori_loop(0, _NCH // 2, body, 0)
        pltpu.sync_copy(out_v, out_hbm.at[pl.ds(wid * _NPW, _NPW)])

    return gmax(table, gidx)


def _tc_pre(h2, x2, Wzr_h, Wzr_x, Wqx):
    def body(h_ref, x_ref, wh_ref, wx_ref, wqx_ref, yzr_ref, yqx_ref):
        yzr_ref[...] = (
            jnp.dot(h_ref[...], wh_ref[...], preferred_element_type=jnp.float32)
            + jnp.dot(x_ref[...], wx_ref[...], preferred_element_type=jnp.float32))
        yqx_ref[...] = jnp.dot(
            x_ref[...], wqx_ref[...], preferred_element_type=jnp.float32)

    return pl.pallas_call(
        body,
        out_shape=(jax.ShapeDtypeStruct((_N, 2 * _HID), jnp.float32),
                   jax.ShapeDtypeStruct((_N, _HID), jnp.float32)),
    )(h2, x2, Wzr_h, Wzr_x, Wqx)


def _tc_mid(mzr, h2, yqx, Wqh, bzr):
    def body(m_ref, h_ref, yqx_ref, wqh_ref, b_ref, z_ref, tq_ref):
        act = jax.nn.sigmoid(m_ref[...] + b_ref[...])
        z = act[:, :_HID]
        r = act[:, _HID:]
        z_ref[...] = z
        tq_ref[...] = yqx_ref[...] + jnp.dot(
            r * h_ref[...], wqh_ref[...], preferred_element_type=jnp.float32)

    return pl.pallas_call(
        body,
        out_shape=(jax.ShapeDtypeStruct((_N, _HID), jnp.float32),
                   jax.ShapeDtypeStruct((_N, _HID), jnp.float32)),
    )(mzr, h2, yqx, Wqh, bzr)


def _tc_post(mq, z, h2, bq1):
    def body(m_ref, z_ref, h_ref, b_ref, out_ref):
        q = jnp.tanh(m_ref[...] + b_ref[...])
        z = z_ref[...]
        out_ref[...] = (1.0 - z) * h_ref[...] + z * q

    return pl.pallas_call(
        body,
        out_shape=jax.ShapeDtypeStruct((_N, _HID), jnp.float32),
    )(mq, z, h2, bq1)


def kernel(h, x, c, graph, Wz, bz, Wr, br, Wq, bq):
    del c
    h2 = h[0]
    x2 = x[0]
    Wzr = jnp.concatenate([Wz, Wr], axis=1)
    Wzr_h, Wzr_x = Wzr[:_HID], Wzr[_HID:]
    Wqh, Wqx = Wq[:_HID], Wq[_HID:]
    bzr = jnp.concatenate([bz, br])[None, :]

    gflat = graph[0].reshape(-1)
    gpad = jnp.concatenate(
        [gflat, jnp.zeros(_NPAD * _K - _N * _K, jnp.int32)])
    gidx = gpad.reshape(_NW, _NCH, _CK)

    yzr, yqx = _tc_pre(h2, x2, Wzr_h, Wzr_x, Wqx)
    mzr = _gather_max(yzr, gidx, 2 * _HID)[:_N]
    z, tq = _tc_mid(mzr, h2, yqx, Wqh, bzr)
    mq = _gather_max(tq, gidx, _HID)[:_N]
    hn = _tc_post(mq, z, h2, bq[None, :])
    return hn[None]


# speedup vs baseline: 13.5558x; 1.6673x over previous
import functools

import jax
import jax.numpy as jnp
from jax import lax
from jax.experimental import pallas as pl
from jax.experimental.pallas import tpu as pltpu
from jax.experimental.pallas import tpu_sc as plsc

_N = 10000
_K = 32
_HID = 64
_IN = 128

_NC, _NS = 2, 16
_NW = _NC * _NS
_NPW = 320
_NPAD = _NW * _NPW
_CPC = 4
_CK = _CPC * _K
_NCH = _NPW // _CPC


def _gather_max(table, gidx, D):
    mesh = plsc.VectorSubcoreMesh(
        core_axis_name="c", subcore_axis_name="s",
        num_cores=_NC, num_subcores=_NS)
    lb = 32

    @functools.partial(
        pl.kernel,
        out_type=jax.ShapeDtypeStruct((_NPAD, D), jnp.bfloat16),
        mesh=mesh,
        compiler_params=pltpu.CompilerParams(use_tc_tiling_on_sc=False),
        scratch_types=[
            pltpu.VMEM((_NCH, _CK), jnp.int32),
            pltpu.VMEM((2, _CK, D), jnp.bfloat16),
            pltpu.VMEM((_NPW, D), jnp.bfloat16),
            pltpu.SemaphoreType.DMA,
            pltpu.SemaphoreType.DMA,
        ],
    )
    def gmax(table_hbm, gidx_hbm, out_hbm, gidx_v, rows_v, out_v, sem0, sem1):
        wid = lax.axis_index("s") * _NC + lax.axis_index("c")
        pltpu.sync_copy(gidx_hbm.at[wid], gidx_v)
        sems = (sem0, sem1)

        def dma(g, b):
            return pltpu.make_async_copy(
                table_hbm.at[gidx_v.at[g]], rows_v.at[b], sems[b])

        dma(0, 0).start()
        dma(1, 1).start()

        def chunk_body(g, b):
            dma(g, b).wait()
            for c in range(_CPC):
                row0 = c * _K
                for dk in range(D // lb):
                    ds = pl.ds(dk * lb, lb)
                    acc = rows_v[b, row0, ds]
                    for k in range(1, _K):
                        acc = jnp.maximum(acc, rows_v[b, row0 + k, ds])
                    out_v[g * _CPC + c, ds] = acc

            @pl.when(g + 2 < _NCH)
            def _():
                dma(g + 2, b).start()

        def body(g2, carry):
            chunk_body(g2 * 2, 0)
            chunk_body(g2 * 2 + 1, 1)
            return carry

        lax.fori_loop(0, _NCH // 2, body, 0)
        pltpu.sync_copy(out_v, out_hbm.at[pl.ds(wid * _NPW, _NPW)])

    return gmax(table, gidx)


def _tc_pre(h2, x2, Wzr_h, Wzr_x, Wqx):
    def body(h_ref, x_ref, wh_ref, wx_ref, wqx_ref, yzr_ref, yqx_ref):
        yzr_ref[...] = (
            jnp.dot(h_ref[...], wh_ref[...], preferred_element_type=jnp.float32)
            + jnp.dot(x_ref[...], wx_ref[...], preferred_element_type=jnp.float32)
        ).astype(jnp.bfloat16)
        yqx_ref[...] = jnp.dot(
            x_ref[...], wqx_ref[...], preferred_element_type=jnp.float32)

    return pl.pallas_call(
        body,
        out_shape=(jax.ShapeDtypeStruct((_N, 2 * _HID), jnp.bfloat16),
                   jax.ShapeDtypeStruct((_N, _HID), jnp.float32)),
    )(h2, x2, Wzr_h, Wzr_x, Wqx)


def _tc_mid(mzr, h2, yqx, Wqh, bzr):
    def body(m_ref, h_ref, yqx_ref, wqh_ref, b_ref, z_ref, tq_ref):
        act = jax.nn.sigmoid(m_ref[...].astype(jnp.float32) + b_ref[...])
        z = act[:, :_HID]
        r = act[:, _HID:]
        z_ref[...] = z
        tq_ref[...] = (yqx_ref[...] + jnp.dot(
            r * h_ref[...], wqh_ref[...], preferred_element_type=jnp.float32)
        ).astype(jnp.bfloat16)

    return pl.pallas_call(
        body,
        out_shape=(jax.ShapeDtypeStruct((_N, _HID), jnp.float32),
                   jax.ShapeDtypeStruct((_N, _HID), jnp.bfloat16)),
    )(mzr, h2, yqx, Wqh, bzr)


def _tc_post(mq, z, h2, bq1):
    def body(m_ref, z_ref, h_ref, b_ref, out_ref):
        q = jnp.tanh(m_ref[...].astype(jnp.float32) + b_ref[...])
        z = z_ref[...]
        out_ref[...] = (1.0 - z) * h_ref[...] + z * q

    return pl.pallas_call(
        body,
        out_shape=jax.ShapeDtypeStruct((_N, _HID), jnp.float32),
    )(mq, z, h2, bq1)


def kernel(h, x, c, graph, Wz, bz, Wr, br, Wq, bq):
    del c
    h2 = h[0]
    x2 = x[0]
    Wzr = jnp.concatenate([Wz, Wr], axis=1)
    Wzr_h, Wzr_x = Wzr[:_HID], Wzr[_HID:]
    Wqh, Wqx = Wq[:_HID], Wq[_HID:]
    bzr = jnp.concatenate([bz, br])[None, :]

    gflat = graph[0].reshape(-1)
    gpad = jnp.concatenate(
        [gflat, jnp.zeros(_NPAD * _K - _N * _K, jnp.int32)])
    gidx = gpad.reshape(_NW, _NCH, _CK)

    yzr, yqx = _tc_pre(h2, x2, Wzr_h, Wzr_x, Wqx)
    mzr = _gather_max(yzr, gidx, 2 * _HID)[:_N]
    z, tq = _tc_mid(mzr, h2, yqx, Wqh, bzr)
    mq = _gather_max(tq, gidx, _HID)[:_N]
    hn = _tc_post(mq, z, h2, bq[None, :])
    return hn[None]


# speedup vs baseline: 30.0076x; 2.2136x over previous
import functools

import jax
import jax.numpy as jnp
from jax import lax
from jax.experimental import pallas as pl
from jax.experimental.pallas import tpu as pltpu
from jax.experimental.pallas import tpu_sc as plsc

_N = 10000
_K = 32
_HID = 64
_IN = 128

_NC, _NS = 2, 16
_NW = _NC * _NS
_NPW = 320
_NPAD = _NW * _NPW
_CPC = 4
_CK = _CPC * _K
_NCH = _NPW // _CPC


def _gather_max(table, gidx, D):
    mesh = plsc.VectorSubcoreMesh(
        core_axis_name="c", subcore_axis_name="s",
        num_cores=_NC, num_subcores=_NS)
    lb = 32

    @functools.partial(
        pl.kernel,
        out_type=jax.ShapeDtypeStruct((_NPAD, D), jnp.bfloat16),
        mesh=mesh,
        compiler_params=pltpu.CompilerParams(use_tc_tiling_on_sc=False),
        scratch_types=[
            pltpu.VMEM((_NCH, _CK), jnp.int32),
            pltpu.VMEM((2, _CK, D), jnp.bfloat16),
            pltpu.VMEM((_NPW, D), jnp.bfloat16),
            pltpu.VMEM_SHARED((_N, D), jnp.bfloat16),
            pltpu.SemaphoreType.DMA,
            pltpu.SemaphoreType.DMA,
        ],
    )
    def gmax(table_hbm, gidx_hbm, out_hbm, gidx_v, rows_v, out_v, table_sh,
             sem0, sem1):
        wid = lax.axis_index("s") * _NC + lax.axis_index("c")
        @pl.when(lax.axis_index("s") == 0)
        def _():
            pltpu.sync_copy(table_hbm, table_sh)

        pltpu.sync_copy(gidx_hbm.at[wid], gidx_v)
        plsc.subcore_barrier()
        sems = (sem0, sem1)

        def dma(g, b):
            return pltpu.make_async_copy(
                table_sh.at[gidx_v.at[g]], rows_v.at[b], sems[b])

        dma(0, 0).start()
        dma(1, 1).start()

        def chunk_body(g, b):
            dma(g, b).wait()
            for c in range(_CPC):
                row0 = c * _K
                for dk in range(D // lb):
                    ds = pl.ds(dk * lb, lb)
                    acc = rows_v[b, row0, ds]
                    for k in range(1, _K):
                        acc = jnp.maximum(acc, rows_v[b, row0 + k, ds])
                    out_v[g * _CPC + c, ds] = acc

            @pl.when(g + 2 < _NCH)
            def _():
                dma(g + 2, b).start()

        def body(g2, carry):
            chunk_body(g2 * 2, 0)
            chunk_body(g2 * 2 + 1, 1)
            return carry

        lax.fori_loop(0, _NCH // 2, body, 0)
        pltpu.sync_copy(out_v, out_hbm.at[pl.ds(wid * _NPW, _NPW)])

    return gmax(table, gidx)


def _tc_pre(h2, x2, Wzr_h, Wzr_x, Wqx):
    def body(h_ref, x_ref, wh_ref, wx_ref, wqx_ref, yzr_ref, yqx_ref):
        yzr_ref[...] = (
            jnp.dot(h_ref[...], wh_ref[...], preferred_element_type=jnp.float32)
            + jnp.dot(x_ref[...], wx_ref[...], preferred_element_type=jnp.float32)
        ).astype(jnp.bfloat16)
        yqx_ref[...] = jnp.dot(
            x_ref[...], wqx_ref[...], preferred_element_type=jnp.float32)

    return pl.pallas_call(
        body,
        out_shape=(jax.ShapeDtypeStruct((_N, 2 * _HID), jnp.bfloat16),
                   jax.ShapeDtypeStruct((_N, _HID), jnp.float32)),
    )(h2, x2, Wzr_h, Wzr_x, Wqx)


def _tc_mid(mzr, h2, yqx, Wqh, bzr):
    def body(m_ref, h_ref, yqx_ref, wqh_ref, b_ref, z_ref, tq_ref):
        act = jax.nn.sigmoid(m_ref[...].astype(jnp.float32) + b_ref[...])
        z = act[:, :_HID]
        r = act[:, _HID:]
        z_ref[...] = z
        tq_ref[...] = (yqx_ref[...] + jnp.dot(
            r * h_ref[...], wqh_ref[...], preferred_element_type=jnp.float32)
        ).astype(jnp.bfloat16)

    return pl.pallas_call(
        body,
        out_shape=(jax.ShapeDtypeStruct((_N, _HID), jnp.float32),
                   jax.ShapeDtypeStruct((_N, _HID), jnp.bfloat16)),
    )(mzr, h2, yqx, Wqh, bzr)


def _tc_post(mq, z, h2, bq1):
    def body(m_ref, z_ref, h_ref, b_ref, out_ref):
        q = jnp.tanh(m_ref[...].astype(jnp.float32) + b_ref[...])
        z = z_ref[...]
        out_ref[...] = (1.0 - z) * h_ref[...] + z * q

    return pl.pallas_call(
        body,
        out_shape=jax.ShapeDtypeStruct((_N, _HID), jnp.float32),
    )(mq, z, h2, bq1)


def kernel(h, x, c, graph, Wz, bz, Wr, br, Wq, bq):
    del c
    h2 = h[0]
    x2 = x[0]
    Wzr = jnp.concatenate([Wz, Wr], axis=1)
    Wzr_h, Wzr_x = Wzr[:_HID], Wzr[_HID:]
    Wqh, Wqx = Wq[:_HID], Wq[_HID:]
    bzr = jnp.concatenate([bz, br])[None, :]

    gflat = graph[0].reshape(-1)
    gpad = jnp.concatenate(
        [gflat, jnp.zeros(_NPAD * _K - _N * _K, jnp.int32)])
    gidx = gpad.reshape(_NW, _NCH, _CK)

    yzr, yqx = _tc_pre(h2, x2, Wzr_h, Wzr_x, Wqx)
    mzr = _gather_max(yzr, gidx, 2 * _HID)[:_N]
    z, tq = _tc_mid(mzr, h2, yqx, Wqh, bzr)
    mq = _gather_max(tq, gidx, _HID)[:_N]
    hn = _tc_post(mq, z, h2, bq[None, :])
    return hn[None]
